# Initial kernel scaffold; baseline (speedup 1.0000x reference)
import jax
import jax.numpy as jnp
from jax.experimental import pallas as pl


def kernel(data, ivectors_weight):
    raise NotImplementedError("write your pallas kernel here")


# speedup vs baseline: 1.6847x; 1.6847x over previous
import functools

import jax
import jax.numpy as jnp
from jax import lax
from jax.experimental import pallas as pl
from jax.experimental.pallas import tpu as pltpu
from jax.experimental.pallas import tpu_sc as plsc

VOCAB = 1000000
EMB = 64
ROWS = 16384
COLS = 50
B = ROWS * COLS
NC = 2
NS = 16
NW = NC * NS
B_PER_W = B // NW
CHUNK = 128
STEPS = B_PER_W // CHUNK

_mesh = plsc.VectorSubcoreMesh(core_axis_name="c", subcore_axis_name="s")


@functools.partial(
    pl.kernel,
    out_type=jax.ShapeDtypeStruct((B, EMB), jnp.float32),
    mesh=_mesh,
    scratch_types=[
        pltpu.VMEM((STEPS, CHUNK), jnp.int32),
        pltpu.VMEM((CHUNK, EMB), jnp.float32),
        pltpu.SemaphoreType.DMA,
    ],
    compiler_params=pltpu.CompilerParams(use_tc_tiling_on_sc=False),
)
def _gather_kernel(idx_hbm, table_hbm, out_hbm, idx_v, rows_v, sem):
    wid = lax.axis_index("s") * NC + lax.axis_index("c")
    base = wid * B_PER_W
    pltpu.sync_copy(idx_hbm.at[wid], idx_v)

    @pl.loop(0, STEPS)
    def _(j):
        pltpu.async_copy(table_hbm.at[idx_v.at[j]], rows_v, sem).wait()
        pltpu.sync_copy(rows_v, out_hbm.at[pl.ds(base + j * CHUNK, CHUNK)])


def kernel(data, ivectors_weight):
    idx = data.reshape(NW, STEPS, CHUNK).astype(jnp.int32)
    out = _gather_kernel(idx, ivectors_weight)
    return out.reshape(ROWS, COLS, EMB)


# speedup vs baseline: 1.8746x; 1.1128x over previous
import functools

import jax
import jax.numpy as jnp
from jax import lax
from jax.experimental import pallas as pl
from jax.experimental.pallas import tpu as pltpu
from jax.experimental.pallas import tpu_sc as plsc

VOCAB = 1000000
EMB = 64
ROWS = 16384
COLS = 50
B = ROWS * COLS
NC = 2
NS = 16
NW = NC * NS
B_PER_W = B // NW
CHUNK = 128
STEPS = B_PER_W // CHUNK
NBUF = 8

_mesh = plsc.VectorSubcoreMesh(core_axis_name="c", subcore_axis_name="s")


@functools.partial(
    pl.kernel,
    out_type=jax.ShapeDtypeStruct((B, EMB), jnp.float32),
    mesh=_mesh,
    scratch_types=(
        [pltpu.VMEM((STEPS, CHUNK), jnp.int32)]
        + [pltpu.VMEM((CHUNK, EMB), jnp.float32) for _ in range(NBUF)]
        + [pltpu.SemaphoreType.DMA for _ in range(2 * NBUF)]
    ),
    compiler_params=pltpu.CompilerParams(use_tc_tiling_on_sc=False),
)
def _gather_kernel(idx_hbm, table_hbm, out_hbm, idx_v, *rest):
    bufs = rest[:NBUF]
    gsems = rest[NBUF:2 * NBUF]
    ssems = rest[2 * NBUF:3 * NBUF]

    wid = lax.axis_index("s") * NC + lax.axis_index("c")
    base = wid * B_PER_W
    pltpu.sync_copy(idx_hbm.at[wid], idx_v)

    def gather(b, j):
        return pltpu.make_async_copy(table_hbm.at[idx_v.at[j]], bufs[b], gsems[b])

    def store(b, j):
        return pltpu.make_async_copy(
            bufs[b], out_hbm.at[pl.ds(base + j * CHUNK, CHUNK)], ssems[b])

    for b in range(NBUF):
        gather(b, b).start()

    @pl.loop(0, STEPS - NBUF, step=NBUF)
    def _(g):
        for b in range(NBUF):
            j = g + b
            gather(b, j).wait()
            store(b, j).start()
        for b in range(NBUF):
            j = g + b
            store(b, j).wait()
            gather(b, j + NBUF).start()

    g_last = STEPS - NBUF
    for b in range(NBUF):
        j = g_last + b
        gather(b, j).wait()
        store(b, j).start()
    for b in range(NBUF):
        store(b, g_last + b).wait()


def kernel(data, ivectors_weight):
    idx = data.reshape(NW, STEPS, CHUNK).astype(jnp.int32)
    out = _gather_kernel(idx, ivectors_weight)
    return out.reshape(ROWS, COLS, EMB)
